# Initial kernel scaffold; baseline (speedup 1.0000x reference)
import functools

import jax
import jax.numpy as jnp
from jax import lax
from jax.experimental import pallas as pl
from jax.experimental.pallas import tpu as pltpu
from jax.experimental.pallas import tpu_sc as plsc

_N = 100000
_N_OPS = 1000
_ROW = 25
_L = 16

_info = plsc.get_sparse_core_info()
_NW = _info.num_cores * _info.num_subcores
_RPW = ((_N + _NW - 1) // _NW + _L - 1) // _L * _L
_GROUPS = _RPW // _L


def _fold_kernel(op_table_ref, op_mlp_W_ref, op_mlp_b_ref,
                 shape_W_ref, shape_b_ref, shape_mlp_W_ref, shape_mlp_b_ref,
                 attr_W_ref, attr_b_ref, attr_mlp_W_ref, attr_mlp_b_ref,
                 ft_ref, wvec_ref):
    opw = op_mlp_W_ref[...]
    ft = jnp.sum(op_table_ref[...] * opw, axis=1, keepdims=True)
    ft_ref[...] = ft + op_mlp_b_ref[...][0]
    w_s = jnp.dot(shape_mlp_W_ref[...], shape_W_ref[...],
                  preferred_element_type=jnp.float32)
    b_s = jnp.sum(shape_mlp_W_ref[...][0] * shape_b_ref[...]) + shape_mlp_b_ref[...]
    w_a = jnp.dot(attr_mlp_W_ref[...], attr_W_ref[...],
                  preferred_element_type=jnp.float32)
    b_a = jnp.sum(attr_mlp_W_ref[...][0] * attr_b_ref[...]) + attr_mlp_b_ref[...]
    wvec_ref[...] = jnp.concatenate(
        [w_s[0], w_a[0], b_s, b_a, jnp.zeros((6,), jnp.float32)])


_mesh = plsc.VectorSubcoreMesh(core_axis_name="c", subcore_axis_name="s")


@functools.partial(
    pl.kernel,
    out_type=jax.ShapeDtypeStruct((_N, 3), jnp.float32),
    mesh=_mesh,
    scratch_types=[
        pltpu.VMEM((_RPW, _ROW), jnp.float32),
        pltpu.VMEM((_RPW, 3), jnp.float32),
        pltpu.VMEM((_N_OPS, 1), jnp.float32),
        pltpu.VMEM((32,), jnp.float32),
    ],
)
def _sc_embed(geo_hbm, ft_hbm, wv_hbm, out_hbm, geo_v, out_v, ft_v, wv_v):
    wid = lax.axis_index("s") * _info.num_cores + lax.axis_index("c")
    base = jnp.minimum(wid * _RPW, _N - _RPW)
    pltpu.sync_copy(geo_hbm.at[pl.ds(base, _RPW)], geo_v)
    pltpu.sync_copy(ft_hbm, ft_v)
    pltpu.sync_copy(wv_hbm, wv_v)

    lanes = lax.iota(jnp.int32, _L)
    zeros = jnp.zeros((_L,), jnp.int32)
    wspl = [plsc.load_gather(wv_v, [jnp.full((_L,), c, jnp.int32)])
            for c in range(26)]

    def body(g, carry):
        rows = g * _L + lanes
        op_idx = plsc.load_gather(geo_v, [rows, zeros]).astype(jnp.int32)
        op_val = plsc.load_gather(ft_v, [op_idx, zeros])
        acc_s = wspl[24]
        for c in range(8):
            col = plsc.load_gather(geo_v, [rows, jnp.full((_L,), 1 + c, jnp.int32)])
            acc_s = acc_s + col * wspl[c]
        acc_a = wspl[25]
        for c in range(16):
            col = plsc.load_gather(geo_v, [rows, jnp.full((_L,), 9 + c, jnp.int32)])
            acc_a = acc_a + col * wspl[8 + c]
        plsc.store_scatter(out_v, [rows, zeros], jnp.abs(op_val))
        plsc.store_scatter(out_v, [rows, jnp.full((_L,), 1, jnp.int32)], jnp.abs(acc_s))
        plsc.store_scatter(out_v, [rows, jnp.full((_L,), 2, jnp.int32)], jnp.abs(acc_a))
        return carry

    lax.fori_loop(0, _GROUPS, body, jnp.int32(0))
    pltpu.sync_copy(out_v, out_hbm.at[pl.ds(base, _RPW)])


def kernel(geo_x, op_table, shape_W, shape_b, attr_W, attr_b,
           op_mlp_W, op_mlp_b, shape_mlp_W, shape_mlp_b,
           attr_mlp_W, attr_mlp_b):
    ft, wvec = pl.pallas_call(
        _fold_kernel,
        out_shape=(jax.ShapeDtypeStruct((_N_OPS, 1), jnp.float32),
                   jax.ShapeDtypeStruct((32,), jnp.float32)),
    )(op_table, op_mlp_W, op_mlp_b, shape_W, shape_b, shape_mlp_W,
      shape_mlp_b, attr_W, attr_b, attr_mlp_W, attr_mlp_b)
    return _sc_embed(geo_x, ft, wvec)


# speedup vs baseline: 1.6072x; 1.6072x over previous
import functools

import jax
import jax.numpy as jnp
from jax import lax
from jax.experimental import pallas as pl
from jax.experimental.pallas import tpu as pltpu
from jax.experimental.pallas import tpu_sc as plsc

_N = 100000
_N_OPS = 1000
_ROW = 25
_L = 16

_info = plsc.get_sparse_core_info()
_NW = _info.num_cores * _info.num_subcores
_RPW = ((_N + _NW - 1) // _NW + _L - 1) // _L * _L
_GROUPS = _RPW // _L


def _fold_kernel(op_table_ref, op_mlp_W_ref, op_mlp_b_ref,
                 shape_W_ref, shape_b_ref, shape_mlp_W_ref, shape_mlp_b_ref,
                 attr_W_ref, attr_b_ref, attr_mlp_W_ref, attr_mlp_b_ref,
                 ft_ref, wvec_ref):
    opw = op_mlp_W_ref[...]
    ft = jnp.sum(op_table_ref[...] * opw, axis=1, keepdims=True)
    ft_ref[...] = ft + op_mlp_b_ref[...][0]
    w_s = jnp.dot(shape_mlp_W_ref[...], shape_W_ref[...],
                  preferred_element_type=jnp.float32)
    b_s = jnp.sum(shape_mlp_W_ref[...][0] * shape_b_ref[...]) + shape_mlp_b_ref[...]
    w_a = jnp.dot(attr_mlp_W_ref[...], attr_W_ref[...],
                  preferred_element_type=jnp.float32)
    b_a = jnp.sum(attr_mlp_W_ref[...][0] * attr_b_ref[...]) + attr_mlp_b_ref[...]
    wvec_ref[...] = jnp.concatenate(
        [w_s[0], w_a[0], b_s, b_a, jnp.zeros((6,), jnp.float32)]).reshape(1, 32)


_mesh = plsc.VectorSubcoreMesh(core_axis_name="c", subcore_axis_name="s")


def _sc_body(geo_hbm, ft_hbm, wv_hbm, out_hbm, geo_v, out_v, ft_v, wv_v):
    wid = lax.axis_index("s") * _info.num_cores + lax.axis_index("c")
    base = jnp.minimum(wid * _RPW, _N - _RPW)
    pltpu.sync_copy(geo_hbm.at[pl.ds(base * _ROW, _RPW * _ROW)], geo_v)
    pltpu.sync_copy(ft_hbm, ft_v)
    pltpu.sync_copy(wv_hbm, wv_v)

    lanes = lax.iota(jnp.int32, _L)
    probe = plsc.load_gather(geo_v, [jnp.zeros((_L,), jnp.int32)])
    zdep = jnp.minimum(probe.astype(jnp.int32), 0)
    wspl = [plsc.load_gather(wv_v, [jnp.full((_L,), c, jnp.int32) + zdep])
            for c in range(26)]

    def body(g, carry):
        rows = g * _L + lanes
        gbase = rows * _ROW
        obase = rows * 3
        op_idx = plsc.load_gather(geo_v, [gbase]).astype(jnp.int32)
        op_val = plsc.load_gather(ft_v, [op_idx])
        acc_s = wspl[24]
        for c in range(8):
            col = plsc.load_gather(geo_v, [gbase + (1 + c)])
            acc_s = acc_s + col * wspl[c]
        acc_a = wspl[25]
        for c in range(16):
            col = plsc.load_gather(geo_v, [gbase + (9 + c)])
            acc_a = acc_a + col * wspl[8 + c]
        plsc.store_scatter(out_v, [obase], jnp.abs(op_val))
        plsc.store_scatter(out_v, [obase + 1], jnp.abs(acc_s))
        plsc.store_scatter(out_v, [obase + 2], jnp.abs(acc_a))
        return carry

    lax.fori_loop(0, _GROUPS, body, jnp.int32(0))
    pltpu.sync_copy(out_v, out_hbm.at[pl.ds(base * 3, _RPW * 3)])


def _make_sc_embed(interpret=False):
    return pl.kernel(
        _sc_body,
        out_type=jax.ShapeDtypeStruct((_N * 3,), jnp.float32),
        mesh=_mesh,
        compiler_params=pltpu.CompilerParams(needs_layout_passes=False,
                                             use_tc_tiling_on_sc=False),
        scratch_types=[
            pltpu.VMEM((_RPW * _ROW,), jnp.float32),
            pltpu.VMEM((_RPW * 3,), jnp.float32),
            pltpu.VMEM((_N_OPS,), jnp.float32),
            pltpu.VMEM((32,), jnp.float32),
        ],
        interpret=interpret,
    )


_sc_embed = _make_sc_embed()


def kernel(geo_x, op_table, shape_W, shape_b, attr_W, attr_b,
           op_mlp_W, op_mlp_b, shape_mlp_W, shape_mlp_b,
           attr_mlp_W, attr_mlp_b):
    ft, wvec = pl.pallas_call(
        _fold_kernel,
        out_shape=(jax.ShapeDtypeStruct((_N_OPS, 1), jnp.float32),
                   jax.ShapeDtypeStruct((1, 32), jnp.float32)),
    )(op_table, op_mlp_W, op_mlp_b, shape_W, shape_b, shape_mlp_W,
      shape_mlp_b, attr_W, attr_b, attr_mlp_W, attr_mlp_b)
    out_flat = _sc_embed(geo_x.reshape(-1), ft.reshape(-1), wvec.reshape(-1))
    return out_flat.reshape(_N, 3)


# speedup vs baseline: 4.8903x; 3.0427x over previous
import functools

import jax
import jax.numpy as jnp
from jax import lax
from jax.experimental import pallas as pl
from jax.experimental.pallas import tpu as pltpu
from jax.experimental.pallas import tpu_sc as plsc

_N = 100000
_N_OPS = 1000
_ROW = 25
_L = 16

_info = plsc.get_sparse_core_info()
_NW = _info.num_cores * _info.num_subcores
_RPW = ((_N + _NW - 1) // _NW + _L - 1) // _L * _L
_GROUPS = _RPW // _L


def _fold_kernel(op_table_ref, op_mlp_W_ref, op_mlp_b_ref,
                 shape_W_ref, shape_b_ref, shape_mlp_W_ref, shape_mlp_b_ref,
                 attr_W_ref, attr_b_ref, attr_mlp_W_ref, attr_mlp_b_ref,
                 ft_ref, wvec_ref):
    opw = op_mlp_W_ref[...]
    ft = jnp.sum(op_table_ref[...] * opw, axis=1, keepdims=True)
    ft_ref[...] = ft + op_mlp_b_ref[...][0]
    w_s = jnp.dot(shape_mlp_W_ref[...], shape_W_ref[...],
                  preferred_element_type=jnp.float32)
    b_s = jnp.sum(shape_mlp_W_ref[...][0] * shape_b_ref[...]) + shape_mlp_b_ref[...]
    w_a = jnp.dot(attr_mlp_W_ref[...], attr_W_ref[...],
                  preferred_element_type=jnp.float32)
    b_a = jnp.sum(attr_mlp_W_ref[...][0] * attr_b_ref[...]) + attr_mlp_b_ref[...]
    wvec_ref[...] = jnp.concatenate(
        [w_s[0], w_a[0], b_s, b_a, jnp.zeros((6,), jnp.float32)]).reshape(1, 32)


_mesh = plsc.VectorSubcoreMesh(core_axis_name="c", subcore_axis_name="s")


def _sc_body(geo_hbm, ft_hbm, wv_hbm, out_hbm, geo_v, out_v, ft_v, wv_v):
    wid = lax.axis_index("s") * _info.num_cores + lax.axis_index("c")
    base = jnp.minimum(wid * _RPW, _N - _RPW)
    pltpu.sync_copy(wv_hbm, wv_v)
    pltpu.sync_copy(ft_hbm, ft_v)
    pltpu.sync_copy(geo_hbm.at[:, pl.ds(base, _RPW)], geo_v)

    probe = geo_v[0, pl.ds(0, _L)]
    zdep = jnp.minimum(probe.astype(jnp.int32), 0)
    wspl = [plsc.load_gather(wv_v, [jnp.full((_L,), c, jnp.int32) + zdep])
            for c in range(26)]

    def body(g, carry):
        sl = pl.ds(g * _L, _L)
        op_idx = geo_v[0, sl].astype(jnp.int32)
        op_val = plsc.load_gather(ft_v, [op_idx])
        acc_s = wspl[24]
        for c in range(8):
            acc_s = acc_s + geo_v[1 + c, sl] * wspl[c]
        acc_a = wspl[25]
        for c in range(16):
            acc_a = acc_a + geo_v[9 + c, sl] * wspl[8 + c]
        out_v[0, sl] = jnp.abs(op_val)
        out_v[1, sl] = jnp.abs(acc_s)
        out_v[2, sl] = jnp.abs(acc_a)
        return carry

    lax.fori_loop(0, _GROUPS, body, jnp.int32(0))
    pltpu.sync_copy(out_v, out_hbm.at[:, pl.ds(base, _RPW)])


def _make_sc_embed(interpret=False):
    return pl.kernel(
        _sc_body,
        out_type=jax.ShapeDtypeStruct((3, _N), jnp.float32),
        mesh=_mesh,
        compiler_params=pltpu.CompilerParams(needs_layout_passes=False,
                                             use_tc_tiling_on_sc=False),
        scratch_types=[
            pltpu.VMEM((_ROW, _RPW), jnp.float32),
            pltpu.VMEM((3, _RPW), jnp.float32),
            pltpu.VMEM((_N_OPS,), jnp.float32),
            pltpu.VMEM((32,), jnp.float32),
        ],
        interpret=interpret,
    )


_sc_embed = _make_sc_embed()


def kernel(geo_x, op_table, shape_W, shape_b, attr_W, attr_b,
           op_mlp_W, op_mlp_b, shape_mlp_W, shape_mlp_b,
           attr_mlp_W, attr_mlp_b):
    ft, wvec = pl.pallas_call(
        _fold_kernel,
        out_shape=(jax.ShapeDtypeStruct((_N_OPS, 1), jnp.float32),
                   jax.ShapeDtypeStruct((1, 32), jnp.float32)),
    )(op_table, op_mlp_W, op_mlp_b, shape_W, shape_b, shape_mlp_W,
      shape_mlp_b, attr_W, attr_b, attr_mlp_W, attr_mlp_b)
    out_planes = _sc_embed(geo_x.T, ft.reshape(-1), wvec.reshape(-1))
    return out_planes.T


# speedup vs baseline: 5.5967x; 1.1445x over previous
import functools

import jax
import jax.numpy as jnp
from jax import lax
from jax.experimental import pallas as pl
from jax.experimental.pallas import tpu as pltpu
from jax.experimental.pallas import tpu_sc as plsc

_N = 100000
_N_OPS = 1000
_ROW = 25
_L = 16

_info = plsc.get_sparse_core_info()
_NW = _info.num_cores * _info.num_subcores
_RPW = ((_N + _NW - 1) // _NW + _L - 1) // _L * _L
_GROUPS = _RPW // _L


def _fold_kernel(op_tableT_ref, op_mlp_W_ref, op_mlp_b_ref,
                 shape_WT_ref, shape_b_ref, shape_mlp_W_ref, shape_mlp_b_ref,
                 attr_WT_ref, attr_b_ref, attr_mlp_W_ref, attr_mlp_b_ref,
                 ft_ref, wvec_ref):
    ft = jnp.dot(op_mlp_W_ref[...], op_tableT_ref[...],
                 preferred_element_type=jnp.float32)
    ft_ref[...] = ft + op_mlp_b_ref[...][0]
    smlp = shape_mlp_W_ref[...]
    w_s = jnp.sum(shape_WT_ref[...] * smlp, axis=1)
    b_s = jnp.sum(smlp[0] * shape_b_ref[...]) + shape_mlp_b_ref[...]
    amlp = attr_mlp_W_ref[...]
    w_a = jnp.sum(attr_WT_ref[...] * amlp, axis=1)
    b_a = jnp.sum(amlp[0] * attr_b_ref[...]) + attr_mlp_b_ref[...]
    wvec_ref[...] = jnp.concatenate(
        [w_s, w_a, b_s, b_a, jnp.zeros((6,), jnp.float32)]).reshape(1, 32)


_mesh = plsc.VectorSubcoreMesh(core_axis_name="c", subcore_axis_name="s")


def _sc_body(geo_hbm, ft_hbm, wv_hbm, out_hbm, geo_v, out_v, ft_v, wv_v):
    wid = lax.axis_index("s") * _info.num_cores + lax.axis_index("c")
    base = jnp.minimum(wid * _RPW, _N - _RPW)
    pltpu.sync_copy(wv_hbm, wv_v)
    pltpu.sync_copy(ft_hbm, ft_v)
    pltpu.sync_copy(geo_hbm.at[:, pl.ds(base, _RPW)], geo_v)

    probe = geo_v[0, pl.ds(0, _L)]
    zdep = jnp.minimum(probe.astype(jnp.int32), 0)
    wspl = [plsc.load_gather(wv_v, [jnp.full((_L,), c, jnp.int32) + zdep])
            for c in range(26)]

    def body(g, carry):
        sl = pl.ds(g * _L, _L)
        op_idx = geo_v[0, sl].astype(jnp.int32)
        op_val = plsc.load_gather(ft_v, [op_idx])
        acc_s = wspl[24]
        for c in range(8):
            acc_s = acc_s + geo_v[1 + c, sl] * wspl[c]
        acc_a = wspl[25]
        for c in range(16):
            acc_a = acc_a + geo_v[9 + c, sl] * wspl[8 + c]
        out_v[0, sl] = jnp.abs(op_val)
        out_v[1, sl] = jnp.abs(acc_s)
        out_v[2, sl] = jnp.abs(acc_a)
        return carry

    lax.fori_loop(0, _GROUPS, body, jnp.int32(0))
    pltpu.sync_copy(out_v, out_hbm.at[:, pl.ds(base, _RPW)])


def _make_sc_embed(interpret=False):
    return pl.kernel(
        _sc_body,
        out_type=jax.ShapeDtypeStruct((3, _N), jnp.float32),
        mesh=_mesh,
        compiler_params=pltpu.CompilerParams(needs_layout_passes=False,
                                             use_tc_tiling_on_sc=False),
        scratch_types=[
            pltpu.VMEM((_ROW, _RPW), jnp.float32),
            pltpu.VMEM((3, _RPW), jnp.float32),
            pltpu.VMEM((_N_OPS,), jnp.float32),
            pltpu.VMEM((32,), jnp.float32),
        ],
        interpret=interpret,
    )


_sc_embed = _make_sc_embed()


def kernel(geo_x, op_table, shape_W, shape_b, attr_W, attr_b,
           op_mlp_W, op_mlp_b, shape_mlp_W, shape_mlp_b,
           attr_mlp_W, attr_mlp_b):
    ft, wvec = pl.pallas_call(
        _fold_kernel,
        out_shape=(jax.ShapeDtypeStruct((1, _N_OPS), jnp.float32),
                   jax.ShapeDtypeStruct((1, 32), jnp.float32)),
    )(op_table.T, op_mlp_W, op_mlp_b, shape_W.T, shape_b, shape_mlp_W,
      shape_mlp_b, attr_W.T, attr_b, attr_mlp_W, attr_mlp_b)
    out_planes = _sc_embed(geo_x.T, ft.reshape(-1), wvec.reshape(-1))
    return out_planes.T


# speedup vs baseline: 6.1124x; 1.0921x over previous
import functools

import jax
import jax.numpy as jnp
from jax import lax
from jax.experimental import pallas as pl
from jax.experimental.pallas import tpu as pltpu
from jax.experimental.pallas import tpu_sc as plsc

_N = 100000
_N_OPS = 1000
_ROW = 25
_L = 16
_TILE = 128

_info = plsc.get_sparse_core_info()
_NW = _info.num_cores * _info.num_subcores

_NFULL = (_N // _TILE) * _TILE
_NTAIL = _N - _NFULL
_CHT = 11
_CH = _CHT * _TILE
_NCHUNK = _NFULL // _CH
_JMAX = (_NCHUNK + _NW - 1) // _NW
_GROUPS = _CH // _L


def _fold_kernel(op_tableT_ref, op_mlp_W_ref, op_mlp_b_ref,
                 shape_WT_ref, shape_b_ref, shape_mlp_W_ref, shape_mlp_b_ref,
                 attr_WT_ref, attr_b_ref, attr_mlp_W_ref, attr_mlp_b_ref,
                 ft_ref, wvec_ref):
    ft = jnp.dot(op_mlp_W_ref[...], op_tableT_ref[...],
                 preferred_element_type=jnp.float32)
    ft_ref[...] = ft + op_mlp_b_ref[...][0]
    smlp = shape_mlp_W_ref[...]
    w_s = jnp.sum(shape_WT_ref[...] * smlp, axis=1)
    b_s = jnp.sum(smlp[0] * shape_b_ref[...]) + shape_mlp_b_ref[...]
    amlp = attr_mlp_W_ref[...]
    w_a = jnp.sum(attr_WT_ref[...] * amlp, axis=1)
    b_a = jnp.sum(amlp[0] * attr_b_ref[...]) + attr_mlp_b_ref[...]
    wvec_ref[...] = jnp.concatenate(
        [w_s, w_a, b_s, b_a, jnp.zeros((6,), jnp.float32)]).reshape(1, 32)


_mesh = plsc.VectorSubcoreMesh(core_axis_name="c", subcore_axis_name="s")


def _sc_body(geo_hbm, tail_hbm, ft_hbm, wv_hbm, out_hbm, tout_hbm,
             geo_v, out_v, tg_v, tout_v, ft_v, wv_v):
    wid = lax.axis_index("s") * _info.num_cores + lax.axis_index("c")
    pltpu.sync_copy(wv_hbm, wv_v)
    pltpu.sync_copy(ft_hbm, ft_v)

    probe = plsc.load_gather(ft_v, [jnp.zeros((_L,), jnp.int32)])
    zdep = jnp.minimum(jnp.abs(probe).astype(jnp.int32), 0)
    wspl = [plsc.load_gather(wv_v, [jnp.full((_L,), c, jnp.int32) + zdep])
            for c in range(26)]

    def compute(src_v, dst_v, ngroups):
        def body(g, carry):
            sl = pl.ds(g * _L, _L)
            op_idx = src_v[0, sl].astype(jnp.int32)
            op_val = plsc.load_gather(ft_v, [op_idx])
            acc_s = wspl[24]
            for c in range(8):
                acc_s = acc_s + src_v[1 + c, sl] * wspl[c]
            acc_a = wspl[25]
            for c in range(16):
                acc_a = acc_a + src_v[9 + c, sl] * wspl[8 + c]
            dst_v[0, sl] = jnp.abs(op_val)
            dst_v[1, sl] = jnp.abs(acc_s)
            dst_v[2, sl] = jnp.abs(acc_a)
            return carry
        lax.fori_loop(0, ngroups, body, jnp.int32(0))

    def round_(j, carry):
        chunk = jnp.minimum(j * _NW + wid, _NCHUNK - 1)
        base = chunk * _CH
        pltpu.sync_copy(geo_hbm.at[:, pl.ds(base, _CH)], geo_v)
        compute(geo_v, out_v, _GROUPS)
        pltpu.sync_copy(out_v, out_hbm.at[:, pl.ds(base, _CH)])
        return carry

    lax.fori_loop(0, _JMAX, round_, jnp.int32(0))

    @pl.when(wid == _NW - 1)
    def _():
        pltpu.sync_copy(tail_hbm, tg_v)
        compute(tg_v, tout_v, _TILE // _L)
        pltpu.sync_copy(tout_v, tout_hbm)


def _make_sc_embed(interpret=False):
    return pl.kernel(
        _sc_body,
        out_type=(jax.ShapeDtypeStruct((3, _NFULL), jnp.float32),
                  jax.ShapeDtypeStruct((3, _TILE), jnp.float32)),
        mesh=_mesh,
        compiler_params=pltpu.CompilerParams(needs_layout_passes=False,
                                             use_tc_tiling_on_sc=True),
        scratch_types=[
            pltpu.VMEM((_ROW, _CH), jnp.float32),
            pltpu.VMEM((3, _CH), jnp.float32),
            pltpu.VMEM((_ROW, _TILE), jnp.float32),
            pltpu.VMEM((3, _TILE), jnp.float32),
            pltpu.VMEM((_N_OPS,), jnp.float32),
            pltpu.VMEM((32,), jnp.float32),
        ],
        interpret=interpret,
    )


_sc_embed = _make_sc_embed()


def kernel(geo_x, op_table, shape_W, shape_b, attr_W, attr_b,
           op_mlp_W, op_mlp_b, shape_mlp_W, shape_mlp_b,
           attr_mlp_W, attr_mlp_b):
    ft, wvec = pl.pallas_call(
        _fold_kernel,
        out_shape=(jax.ShapeDtypeStruct((1, _N_OPS), jnp.float32),
                   jax.ShapeDtypeStruct((1, 32), jnp.float32)),
    )(op_table.T, op_mlp_W, op_mlp_b, shape_W.T, shape_b, shape_mlp_W,
      shape_mlp_b, attr_W.T, attr_b, attr_mlp_W, attr_mlp_b)
    geoT = geo_x.T
    tail = jnp.pad(lax.slice(geoT, (0, _NFULL), (_ROW, _N)),
                   ((0, 0), (0, _TILE - _NTAIL)))
    out_full, out_tail = _sc_embed(geoT, tail, ft.reshape(-1),
                                   wvec.reshape(-1))
    out = jnp.concatenate([out_full, out_tail[:, :_NTAIL]], axis=1)
    return out.T


# speedup vs baseline: 6.8728x; 1.1244x over previous
import functools

import jax
import jax.numpy as jnp
from jax import lax
from jax.experimental import pallas as pl
from jax.experimental.pallas import tpu as pltpu
from jax.experimental.pallas import tpu_sc as plsc

_N = 100000
_N_OPS = 1000
_ROW = 25
_L = 16
_TILE = 128

_info = plsc.get_sparse_core_info()
_NW = _info.num_cores * _info.num_subcores

_NFULL = (_N // _TILE) * _TILE
_NTAIL = _N - _NFULL
_NT = _NFULL // _TILE
_CHT = ((_NT + _NW - 1) // _NW + 1) // 2 * 2
_CH = _CHT * _TILE
_GROUPS = _CH // _L


def _fold_kernel(op_tableT_ref, op_mlp_W_ref, op_mlp_b_ref,
                 shape_WT_ref, shape_b_ref, shape_mlp_W_ref, shape_mlp_b_ref,
                 attr_WT_ref, attr_b_ref, attr_mlp_W_ref, attr_mlp_b_ref,
                 ft_ref, wvec_ref):
    ft = jnp.dot(op_mlp_W_ref[...], op_tableT_ref[...],
                 preferred_element_type=jnp.float32)
    ft_ref[...] = ft + op_mlp_b_ref[...][0]
    smlp = shape_mlp_W_ref[...]
    w_s = jnp.sum(shape_WT_ref[...] * smlp, axis=1)
    b_s = jnp.sum(smlp[0] * shape_b_ref[...]) + shape_mlp_b_ref[...]
    amlp = attr_mlp_W_ref[...]
    w_a = jnp.sum(attr_WT_ref[...] * amlp, axis=1)
    b_a = jnp.sum(amlp[0] * attr_b_ref[...]) + attr_mlp_b_ref[...]
    wvec_ref[...] = jnp.concatenate(
        [w_s, w_a, b_s, b_a, jnp.zeros((6,), jnp.float32)]).reshape(1, 32)


_mesh = plsc.VectorSubcoreMesh(core_axis_name="c", subcore_axis_name="s")


_HALF = _CH // 2


def _sc_body(geo_hbm, tail_hbm, ft_hbm, wv_hbm, out_hbm, tout_hbm,
             geo_v, out_v, tg_v, tout_v, ft_v, wv_v):
    wid = lax.axis_index("s") * _info.num_cores + lax.axis_index("c")
    pltpu.sync_copy(wv_hbm, wv_v)
    pltpu.sync_copy(ft_hbm, ft_v)

    probe = plsc.load_gather(ft_v, [jnp.zeros((_L,), jnp.int32)])
    zdep = jnp.minimum(jnp.abs(probe).astype(jnp.int32), 0)
    wspl = [plsc.load_gather(wv_v, [jnp.full((_L,), c, jnp.int32) + zdep])
            for c in range(26)]

    def compute(src_v, dst_v, dst_off, ngroups):
        def body(g, carry):
            sl = pl.ds(g * _L, _L)
            op_idx = src_v[0, sl].astype(jnp.int32)
            op_val = plsc.load_gather(ft_v, [op_idx])
            acc_s = wspl[24]
            for c in range(8):
                acc_s = acc_s + src_v[1 + c, sl] * wspl[c]
            acc_a = wspl[25]
            for c in range(16):
                acc_a = acc_a + src_v[9 + c, sl] * wspl[8 + c]
            osl = pl.ds(dst_off + g * _L, _L)
            dst_v[0, osl] = jnp.abs(op_val)
            dst_v[1, osl] = jnp.abs(acc_s)
            dst_v[2, osl] = jnp.abs(acc_a)
            return carry
        lax.fori_loop(0, ngroups, body, jnp.int32(0))

    base = jnp.minimum(wid * _CHT, _NT - _CHT) * _TILE
    for h in range(2):
        pltpu.sync_copy(geo_hbm.at[:, pl.ds(base + h * _HALF, _HALF)], geo_v)
        compute(geo_v, out_v, h * _HALF, _HALF // _L)
    pltpu.sync_copy(out_v, out_hbm.at[:, pl.ds(base, _CH)])

    @pl.when(wid == _NW - 1)
    def _():
        pltpu.sync_copy(tail_hbm, tg_v)
        compute(tg_v, tout_v, 0, _TILE // _L)
        pltpu.sync_copy(tout_v, tout_hbm)


def _make_sc_embed(interpret=False):
    return pl.kernel(
        _sc_body,
        out_type=(jax.ShapeDtypeStruct((3, _NFULL), jnp.float32),
                  jax.ShapeDtypeStruct((3, _TILE), jnp.float32)),
        mesh=_mesh,
        compiler_params=pltpu.CompilerParams(needs_layout_passes=False,
                                             use_tc_tiling_on_sc=True),
        scratch_types=[
            pltpu.VMEM((_ROW, _HALF), jnp.float32),
            pltpu.VMEM((3, _CH), jnp.float32),
            pltpu.VMEM((_ROW, _TILE), jnp.float32),
            pltpu.VMEM((3, _TILE), jnp.float32),
            pltpu.VMEM((_N_OPS,), jnp.float32),
            pltpu.VMEM((32,), jnp.float32),
        ],
        interpret=interpret,
    )


_sc_embed = _make_sc_embed()


def kernel(geo_x, op_table, shape_W, shape_b, attr_W, attr_b,
           op_mlp_W, op_mlp_b, shape_mlp_W, shape_mlp_b,
           attr_mlp_W, attr_mlp_b):
    ft, wvec = pl.pallas_call(
        _fold_kernel,
        out_shape=(jax.ShapeDtypeStruct((1, _N_OPS), jnp.float32),
                   jax.ShapeDtypeStruct((1, 32), jnp.float32)),
    )(op_table.T, op_mlp_W, op_mlp_b, shape_W.T, shape_b, shape_mlp_W,
      shape_mlp_b, attr_W.T, attr_b, attr_mlp_W, attr_mlp_b)
    geoT = geo_x.T
    tail = jnp.pad(lax.slice(geoT, (0, _NFULL), (_ROW, _N)),
                   ((0, 0), (0, _TILE - _NTAIL)))
    out_full, out_tail = _sc_embed(geoT, tail, ft.reshape(-1),
                                   wvec.reshape(-1))
    out = jnp.concatenate([out_full, out_tail[:, :_NTAIL]], axis=1)
    return out.T


# speedup vs baseline: 7.1565x; 1.0413x over previous
import functools

import jax
import jax.numpy as jnp
from jax import lax
from jax.experimental import pallas as pl
from jax.experimental.pallas import tpu as pltpu
from jax.experimental.pallas import tpu_sc as plsc

_N = 100000
_N_OPS = 1000
_ROW = 25
_L = 16
_TILE = 128

_info = plsc.get_sparse_core_info()
_NW = _info.num_cores * _info.num_subcores

_NFULL = (_N // _TILE) * _TILE
_NTAIL = _N - _NFULL
_NT = _NFULL // _TILE
_CHT = ((_NT + _NW - 1) // _NW + 1) // 2 * 2
_CH = _CHT * _TILE
_GROUPS = _CH // _L


def _fold_kernel(op_tableT_ref, op_mlp_W_ref, op_mlp_b_ref,
                 shape_WT_ref, shape_b_ref, shape_mlp_W_ref, shape_mlp_b_ref,
                 attr_WT_ref, attr_b_ref, attr_mlp_W_ref, attr_mlp_b_ref,
                 ft_ref, wvec_ref):
    ft = jnp.dot(op_mlp_W_ref[...], op_tableT_ref[...],
                 preferred_element_type=jnp.float32)
    ft_ref[...] = ft + op_mlp_b_ref[...][0]
    smlp = shape_mlp_W_ref[...]
    w_s = jnp.sum(shape_WT_ref[...] * smlp, axis=1)
    b_s = jnp.sum(smlp[0] * shape_b_ref[...]) + shape_mlp_b_ref[...]
    amlp = attr_mlp_W_ref[...]
    w_a = jnp.sum(attr_WT_ref[...] * amlp, axis=1)
    b_a = jnp.sum(amlp[0] * attr_b_ref[...]) + attr_mlp_b_ref[...]
    wvec_ref[...] = jnp.concatenate(
        [w_s, w_a, b_s, b_a, jnp.zeros((6,), jnp.float32)]).reshape(1, 32)


_mesh = plsc.VectorSubcoreMesh(core_axis_name="c", subcore_axis_name="s")


_STAGES = ((0, 8), (8, 6), (14, 6), (20, 6))
_SBUF = max(s for _, s in _STAGES) * _TILE


def _sc_body(geo_hbm, tail_hbm, ft_hbm, wv_hbm, out_hbm, tout_hbm,
             geo_v, geo_w, out_v, tg_v, tout_v, ft_v, wv_v, sem_a, sem_b):
    wid = lax.axis_index("s") * _info.num_cores + lax.axis_index("c")
    pltpu.sync_copy(wv_hbm, wv_v)
    pltpu.sync_copy(ft_hbm, ft_v)

    probe = plsc.load_gather(ft_v, [jnp.zeros((_L,), jnp.int32)])
    zdep = jnp.minimum(jnp.abs(probe).astype(jnp.int32), 0)
    wspl = [plsc.load_gather(wv_v, [jnp.full((_L,), c, jnp.int32) + zdep])
            for c in range(26)]

    def compute(src_v, dst_v, dst_off, ngroups):
        def body(g, carry):
            sl = pl.ds(g * _L, _L)
            op_idx = src_v[0, sl].astype(jnp.int32)
            op_val = plsc.load_gather(ft_v, [op_idx])
            acc_s = wspl[24]
            for c in range(8):
                acc_s = acc_s + src_v[1 + c, sl] * wspl[c]
            acc_a = wspl[25]
            for c in range(16):
                acc_a = acc_a + src_v[9 + c, sl] * wspl[8 + c]
            osl = pl.ds(dst_off + g * _L, _L)
            dst_v[0, osl] = jnp.abs(op_val)
            dst_v[1, osl] = jnp.abs(acc_s)
            dst_v[2, osl] = jnp.abs(acc_a)
            return carry
        lax.fori_loop(0, ngroups, body, jnp.int32(0))

    base = jnp.minimum(wid * _CHT, _NT - _CHT) * _TILE
    bufs = (geo_v, geo_w)
    sems = (sem_a, sem_b)

    def stage_copy(i):
        off, sz = _STAGES[i]
        return pltpu.async_copy(
            geo_hbm.at[:, pl.ds(base + off * _TILE, sz * _TILE)],
            bufs[i % 2].at[:, pl.ds(0, sz * _TILE)], sems[i % 2])

    pending = stage_copy(0)
    for i in range(len(_STAGES)):
        cur = pending
        if i + 1 < len(_STAGES):
            pending = stage_copy(i + 1)
        cur.wait()
        off, sz = _STAGES[i]
        compute(bufs[i % 2], out_v, off * _TILE, sz * _TILE // _L)
    pltpu.sync_copy(out_v, out_hbm.at[:, pl.ds(base, _CH)])

    @pl.when(wid == _NW - 1)
    def _():
        pltpu.sync_copy(tail_hbm, tg_v)
        compute(tg_v, tout_v, 0, _TILE // _L)
        pltpu.sync_copy(tout_v, tout_hbm)


def _make_sc_embed(interpret=False):
    return pl.kernel(
        _sc_body,
        out_type=(jax.ShapeDtypeStruct((3, _NFULL), jnp.float32),
                  jax.ShapeDtypeStruct((3, _TILE), jnp.float32)),
        mesh=_mesh,
        compiler_params=pltpu.CompilerParams(needs_layout_passes=False,
                                             use_tc_tiling_on_sc=True),
        scratch_types=[
            pltpu.VMEM((_ROW, _SBUF), jnp.float32),
            pltpu.VMEM((_ROW, _SBUF), jnp.float32),
            pltpu.VMEM((3, _CH), jnp.float32),
            pltpu.VMEM((_ROW, _TILE), jnp.float32),
            pltpu.VMEM((3, _TILE), jnp.float32),
            pltpu.VMEM((_N_OPS,), jnp.float32),
            pltpu.VMEM((32,), jnp.float32),
            pltpu.SemaphoreType.DMA,
            pltpu.SemaphoreType.DMA,
        ],
        interpret=interpret,
    )


_sc_embed = _make_sc_embed()


def kernel(geo_x, op_table, shape_W, shape_b, attr_W, attr_b,
           op_mlp_W, op_mlp_b, shape_mlp_W, shape_mlp_b,
           attr_mlp_W, attr_mlp_b):
    ft, wvec = pl.pallas_call(
        _fold_kernel,
        out_shape=(jax.ShapeDtypeStruct((1, _N_OPS), jnp.float32),
                   jax.ShapeDtypeStruct((1, 32), jnp.float32)),
    )(op_table.T, op_mlp_W, op_mlp_b, shape_W.T, shape_b, shape_mlp_W,
      shape_mlp_b, attr_W.T, attr_b, attr_mlp_W, attr_mlp_b)
    geoT = geo_x.T
    tail = jnp.pad(lax.slice(geoT, (0, _NFULL), (_ROW, _N)),
                   ((0, 0), (0, _TILE - _NTAIL)))
    out_full, out_tail = _sc_embed(geoT, tail, ft.reshape(-1),
                                   wvec.reshape(-1))
    out = jnp.concatenate([out_full, out_tail[:, :_NTAIL]], axis=1)
    return out.T
